# Initial kernel scaffold; baseline (speedup 1.0000x reference)
import jax
import jax.numpy as jnp
from jax.experimental import pallas as pl


def kernel(x, y):
    raise NotImplementedError("write your pallas kernel here")


# speedup vs baseline: 1.2413x; 1.2413x over previous
import functools

import jax
import jax.numpy as jnp
import numpy as np
from jax.experimental import pallas as pl

B = 4
N = 4096
NSLOT = 2 * B
NELEM = 3 * N
QB = 256
N_BISECT = 48

_QS = (0.05, 0.95, 0.25, 0.75)
_IDXF = [np.float32(q) * np.float32(NELEM - 1) for q in _QS]
_KLO = [int(np.floor(i)) for i in _IDXF]
_FRAC = [np.float32(i - np.floor(i)) for i in _IDXF]
_RANKS = []
for _k in _KLO:
    _RANKS.extend([_k, _k + 1])


def _stage1_body(qref, ktref, rref):
    q = qref[0]
    kt = ktref[0]
    qx, qy, qz = q[:, 0:1], q[:, 1:2], q[:, 2:3]
    kx, ky, kz = kt[0:1, :], kt[1:2, :], kt[2:3, :]
    n1 = jnp.sum(q * q, axis=1, keepdims=True)
    n2 = jnp.sum(kt * kt, axis=0, keepdims=True)
    dot = jax.lax.dot_general(
        q.astype(jnp.bfloat16), kt.astype(jnp.bfloat16),
        (((1,), (0,)), ((), ())),
        preferred_element_type=jnp.float32)
    d2 = (n1 + n2) - 2.0 * dot
    m = jnp.min(d2, axis=1, keepdims=True)
    ii = jax.lax.broadcasted_iota(jnp.int32, (QB, N), 1)
    idx = jnp.min(jnp.where(d2 == m, ii, N), axis=1, keepdims=True)
    sel = ii == idx
    cx = jnp.sum(jnp.where(sel, kx, 0.0), axis=1, keepdims=True)
    cy = jnp.sum(jnp.where(sel, ky, 0.0), axis=1, keepdims=True)
    cz = jnp.sum(jnp.where(sel, kz, 0.0), axis=1, keepdims=True)
    rref[0] = jnp.concatenate([qx - cx, qy - cy, qz - cz], axis=1)


def _stage2_body(rref, ranks_ref, bref, eref):
    ranks = ranks_ref[...]
    beg_stds = []
    end_stds = []
    for s in range(NSLOT):
        v = rref[s]
        vmin = jnp.min(v)
        vmax = jnp.max(v)
        lo0 = jnp.full((8, 1), vmin - 1.0, jnp.float32)
        hi0 = jnp.full((8, 1), vmax, jnp.float32)

        def body(_, carry, v=v, ranks=ranks):
            lo, hi = carry
            mid = 0.5 * (lo + hi)
            le = (v[None, :, :] <= mid[:, :, None]).astype(jnp.float32)
            cnt = jnp.sum(jnp.sum(le, axis=2), axis=1, keepdims=True)
            pred = cnt >= ranks
            return jnp.where(pred, lo, mid), jnp.where(pred, mid, hi)

        _, o = jax.lax.fori_loop(0, N_BISECT, body, (lo0, hi0))
        qv = [o[2 * j, 0] * (1.0 - float(_FRAC[j])) + o[2 * j + 1, 0] * float(_FRAC[j])
              for j in range(4)]
        q05, q95, q25, q75 = qv
        for thr_mask, acc in (((v < q05) | (v > q95), beg_stds),
                              ((v > q25) & (v < q75), end_stds)):
            m = thr_mask.astype(jnp.float32)
            n = jnp.sum(m)
            mean = jnp.sum(v * m) / n
            var = jnp.sum(((v - mean) ** 2) * m) / (n - 1.0)
            acc.append(jnp.sqrt(var))
    beg = 0.0
    end = 0.0
    for b in range(B):
        beg += jnp.maximum(beg_stds[2 * b], beg_stds[2 * b + 1])
        end += jnp.maximum(end_stds[2 * b], end_stds[2 * b + 1])
    bref[...] = jnp.broadcast_to(beg / B, (1, 1))
    eref[...] = jnp.broadcast_to(end / B, (1, 1))


@jax.jit
def kernel(x, y):
    q_all = jnp.stack([x, y], axis=1).reshape(NSLOT, N, 3)
    k_all = jnp.stack([y, x], axis=1).reshape(NSLOT, N, 3)
    kt_all = k_all.transpose(0, 2, 1)

    resid = pl.pallas_call(
        _stage1_body,
        grid=(NSLOT, N // QB),
        in_specs=[
            pl.BlockSpec((1, QB, 3), lambda s, qb: (s, qb, 0)),
            pl.BlockSpec((1, 3, N), lambda s, qb: (s, 0, 0)),
        ],
        out_specs=pl.BlockSpec((1, QB, 3), lambda s, qb: (s, qb, 0)),
        out_shape=jax.ShapeDtypeStruct((NSLOT, N, 3), jnp.float32),
    )(q_all, kt_all)

    r_flat = resid.reshape(NSLOT, NELEM // 128, 128)
    ranks = jnp.asarray(np.array(_RANKS, np.float32).reshape(8, 1) + 1.0)

    beg, end = pl.pallas_call(
        _stage2_body,
        out_shape=(jax.ShapeDtypeStruct((1, 1), jnp.float32),
                   jax.ShapeDtypeStruct((1, 1), jnp.float32)),
    )(r_flat, ranks)
    return (beg[0, 0], end[0, 0])
